# Initial kernel scaffold; baseline (speedup 1.0000x reference)
import jax
import jax.numpy as jnp
from jax.experimental import pallas as pl


def kernel(x, table):
    raise NotImplementedError("write your pallas kernel here")


# speedup vs baseline: 1.0415x; 1.0415x over previous
import functools

import jax
import jax.numpy as jnp
import numpy as np
from jax import lax
from jax.experimental import pallas as pl
from jax.experimental.pallas import tpu as pltpu
from jax.experimental.pallas import tpu_sc as plsc

_V = 100000
_S = 2048
_D = 1024
_B = 4

_NC, _NS = 2, 16
_NW = _NC * _NS
_POS_PER_W = _S // _NW
_CHUNK = 32
_NPC = _POS_PER_W // _CHUNK
_LANES = 16
_VECS_PER_ROW = _D // _LANES


def _positional_encoding(seq: int, d: int) -> jnp.ndarray:
    pos = np.arange(seq, dtype=np.float32)[:, None]
    i = np.arange(d, dtype=np.float32)[None, :]
    ang = pos / np.power(10000.0, (2.0 * np.floor(i / 2.0)) / float(d))
    pe = np.zeros((seq, d), dtype=np.float32)
    pe[:, 0::2] = np.sin(ang[:, 0::2])
    pe[:, 1::2] = np.cos(ang[:, 1::2])
    return jnp.asarray(pe)


def _add_pe(rows_v, pe_v, pe_row_base):
    @plsc.parallel_loop(0, _CHUNK * _VECS_PER_ROW, 1, unroll=8)
    def _(j):
        r = j // _VECS_PER_ROW
        c = (j % _VECS_PER_ROW) * _LANES
        plsc.addupdate(
            rows_v.at[r, pl.ds(c, _LANES)],
            pe_v[pe_row_base + r, pl.ds(c, _LANES)],
        )


def _body(x_hbm, table_hbm, pe_hbm, out_hbm,
          pe_v, idx_a, idx_b, rows_a, rows_b, sem_a, sem_b):
    wid = lax.axis_index("s") * _NC + lax.axis_index("c")
    pos0 = wid * _POS_PER_W

    units = [(pc, b) for pc in range(_NPC) for b in range(_B)]
    idx_bufs = (idx_a, idx_b)
    row_bufs = (rows_a, rows_b)
    sems = (sem_a, sem_b)

    def start(i):
        pc, b = units[i]
        k = i % 2
        flat = b * _S + pos0 + pc * _CHUNK
        pltpu.sync_copy(x_hbm.at[pl.ds(flat, _CHUNK)], idx_bufs[k])
        return pltpu.async_copy(table_hbm.at[idx_bufs[k]], row_bufs[k], sems[k])

    copies = {0: start(0)}
    for i in range(len(units)):
        pc, b = units[i]
        k = i % 2
        if i + 1 < len(units):
            copies[i + 1] = start(i + 1)
        if i % _B == 0:
            pltpu.sync_copy(pe_hbm.at[pl.ds(pos0 + pc * _CHUNK, _CHUNK)], pe_v)
        copies[i].wait()
        _add_pe(row_bufs[k], pe_v, 0)
        flat = b * _S + pos0 + pc * _CHUNK
        pltpu.sync_copy(row_bufs[k], out_hbm.at[pl.ds(flat, _CHUNK)])


@jax.jit
def _run(x_flat, table, pe):
    mesh = plsc.VectorSubcoreMesh(
        core_axis_name="c", subcore_axis_name="s",
        num_cores=_NC, num_subcores=_NS,
    )
    f = pl.kernel(
        _body,
        out_type=jax.ShapeDtypeStruct((_B * _S, _D), jnp.float32),
        mesh=mesh,
        scratch_types=[
            pltpu.VMEM((_CHUNK, _D), jnp.float32),
            pltpu.VMEM((_CHUNK,), jnp.int32),
            pltpu.VMEM((_CHUNK,), jnp.int32),
            pltpu.VMEM((_CHUNK, _D), jnp.float32),
            pltpu.VMEM((_CHUNK, _D), jnp.float32),
            pltpu.SemaphoreType.DMA,
            pltpu.SemaphoreType.DMA,
        ],
    )
    return f(x_flat, table, pe)


def kernel(x, table):
    pe = _positional_encoding(_S, _D)
    x_flat = x.reshape(-1).astype(jnp.int32)
    out = _run(x_flat, table, pe)
    return out.reshape(_B, _S, _D)
